# baseline (speedup 1.0000x reference)
import functools

import jax
import jax.numpy as jnp
from jax import lax
from jax.experimental import pallas as pl
from jax.experimental.pallas import tpu as pltpu
from jax.experimental.pallas import tpu_sc as plsc

_NC = 2
_NS = 16
_NW = _NC * _NS

_CHUNK = 64


def _gather_kernel(total, dc, n_chunks):
    mesh = plsc.VectorSubcoreMesh(core_axis_name="c", subcore_axis_name="s")
    n_per_w = n_chunks * _CHUNK
    n_pairs = n_chunks // 2

    @functools.partial(
        pl.kernel,
        mesh=mesh,
        out_type=jax.ShapeDtypeStruct((total, dc), jnp.int32),
        scratch_types=[
            pltpu.VMEM((n_chunks, _CHUNK), jnp.int32),
            pltpu.VMEM((_CHUNK, dc), jnp.int32),
            pltpu.VMEM((_CHUNK, dc), jnp.int32),
            pltpu.SemaphoreType.DMA,
            pltpu.SemaphoreType.DMA,
        ],
    )
    def k(pe_hbm, idx_hbm, out_hbm, idx_v, rows0, rows1, gsem0, gsem1):
        wid = lax.axis_index("s") * _NC + lax.axis_index("c")
        base = wid * n_per_w
        pltpu.sync_copy(idx_hbm.at[wid], idx_v)

        def g_start(c, buf, sem):
            pltpu.async_copy(pe_hbm.at[idx_v.at[c]], buf, sem)

        def g_wait(c, buf, sem):
            pltpu.make_async_copy(pe_hbm.at[idx_v.at[c]], buf, sem).wait()

        def put(c, buf):
            pltpu.sync_copy(buf, out_hbm.at[pl.ds(base + c * _CHUNK, _CHUNK)])

        g_start(0, rows0, gsem0)
        g_start(1, rows1, gsem1)

        def body(p, carry):
            c0 = 2 * p
            g_wait(c0, rows0, gsem0)
            put(c0, rows0)
            g_start(c0 + 2, rows0, gsem0)
            g_wait(c0 + 1, rows1, gsem1)
            put(c0 + 1, rows1)
            g_start(c0 + 3, rows1, gsem1)
            return carry

        lax.fori_loop(0, n_pairs - 1, body, 0)

        c0 = n_chunks - 2
        g_wait(c0, rows0, gsem0)
        put(c0, rows0)
        g_wait(c0 + 1, rows1, gsem1)
        put(c0 + 1, rows1)

    return k


def kernel(x, pe):
    batch, seq_len = x.shape
    max_len, d_model = pe.shape
    total = batch * seq_len
    dc = d_model // 2
    n_per_w = total // _NW
    n_chunks = n_per_w // _CHUNK
    idx = x.reshape(_NW, n_chunks, _CHUNK)
    pe_b = pe.astype(jnp.bfloat16).reshape(max_len, dc, 2)
    pe_i = lax.bitcast_convert_type(pe_b, jnp.int32)
    out_i = _gather_kernel(total, dc, n_chunks)(pe_i, idx)
    out_b = lax.bitcast_convert_type(out_i, jnp.bfloat16)
    return out_b.reshape(batch, seq_len, d_model).astype(jnp.float32)


# speedup vs baseline: 6.6764x; 6.6764x over previous
import functools

import jax
import jax.numpy as jnp
from jax import lax
from jax.experimental import pallas as pl
from jax.experimental.pallas import tpu as pltpu
from jax.experimental.pallas import tpu_sc as plsc

_NC = 2
_NS = 16
_NW = _NC * _NS

_CHUNK = 32


def _gather_kernel(total, d_model, n_chunks):
    mesh = plsc.VectorSubcoreMesh(core_axis_name="c", subcore_axis_name="s")
    n_per_w = n_chunks * _CHUNK
    n_pairs = n_chunks // 2

    @functools.partial(
        pl.kernel,
        mesh=mesh,
        out_type=jax.ShapeDtypeStruct((total, d_model), jnp.float32),
        scratch_types=[
            pltpu.VMEM((n_chunks, _CHUNK), jnp.int32),
            pltpu.VMEM((_CHUNK, d_model), jnp.float32),
            pltpu.VMEM((_CHUNK, d_model), jnp.float32),
            pltpu.SemaphoreType.DMA,
            pltpu.SemaphoreType.DMA,
        ],
    )
    def k(pe_hbm, idx_hbm, out_hbm, idx_v, rows0, rows1, gsem0, gsem1):
        wid = lax.axis_index("s") * _NC + lax.axis_index("c")
        base = wid * n_per_w
        pltpu.sync_copy(idx_hbm.at[wid], idx_v)

        def g_start(c, buf, sem):
            pltpu.async_copy(pe_hbm.at[idx_v.at[c]], buf, sem)

        def g_wait(c, buf, sem):
            pltpu.make_async_copy(pe_hbm.at[idx_v.at[c]], buf, sem).wait()

        def put(c, buf):
            pltpu.sync_copy(buf, out_hbm.at[pl.ds(base + c * _CHUNK, _CHUNK)])

        g_start(0, rows0, gsem0)
        g_start(1, rows1, gsem1)

        def body(p, carry):
            c0 = 2 * p
            g_wait(c0, rows0, gsem0)
            put(c0, rows0)
            g_start(c0 + 2, rows0, gsem0)
            g_wait(c0 + 1, rows1, gsem1)
            put(c0 + 1, rows1)
            g_start(c0 + 3, rows1, gsem1)
            return carry

        lax.fori_loop(0, n_pairs - 1, body, 0)

        c0 = n_chunks - 2
        g_wait(c0, rows0, gsem0)
        put(c0, rows0)
        g_wait(c0 + 1, rows1, gsem1)
        put(c0 + 1, rows1)

    return k


def kernel(x, pe):
    batch, seq_len = x.shape
    max_len, d_model = pe.shape
    total = batch * seq_len
    n_per_w = total // _NW
    n_chunks = n_per_w // _CHUNK
    idx = x.reshape(_NW, n_chunks, _CHUNK)
    out = _gather_kernel(total, d_model, n_chunks)(pe, idx)
    return out.reshape(batch, seq_len, d_model)
